# baseline (speedup 1.0000x reference)
import functools

import jax
import jax.numpy as jnp
from jax import lax
from jax.experimental import pallas as pl
from jax.experimental.pallas import tpu as pltpu
from jax.experimental.pallas import tpu_sc as plsc

NCLS = 100
BATCH = 16384
NC = 2
NS = 16
L = 16
NW = NC * NS
CHUNK = BATCH // NW
NV = CHUNK // L
CW_PAD = 112

_LN2_HI = 0.693359375
_LN2_LO = -2.12194440e-4
_SQRTHF = 0.70710678118654752440


def _log_f32(x):
    bits = plsc.bitcast(x, jnp.int32)
    e = (bits >> 23) - 126
    m = plsc.bitcast(
        (bits & jnp.int32(0x007FFFFF)) | jnp.int32(0x3F000000), jnp.float32
    )
    below = m < _SQRTHF
    e = jnp.where(below, e - 1, e)
    t = jnp.where(below, m + m - 1.0, m - 1.0)
    ef = e.astype(jnp.float32)

    z = t * t
    p = jnp.full((L,), 7.0376836292e-2, jnp.float32)
    p = p * t - 1.1514610310e-1
    p = p * t + 1.1676998740e-1
    p = p * t - 1.2420140846e-1
    p = p * t + 1.4249322787e-1
    p = p * t - 1.6668057665e-1
    p = p * t + 2.0000714765e-1
    p = p * t - 2.4999993993e-1
    p = p * t + 3.3333331174e-1
    y = t * z * p
    y = y + ef * _LN2_LO
    y = y - 0.5 * z
    return t + y + ef * _LN2_HI


_MESH = plsc.VectorSubcoreMesh(core_axis_name="c", subcore_axis_name="s")


@functools.partial(
    pl.kernel,
    mesh=_MESH,
    compiler_params=pltpu.CompilerParams(
        needs_layout_passes=False, use_tc_tiling_on_sc=True
    ),
    out_type=jax.ShapeDtypeStruct((NW, L), jnp.float32),
    scratch_types=[
        pltpu.VMEM((CHUNK,), jnp.int32),
        pltpu.VMEM((NCLS, CHUNK), jnp.float32),
        pltpu.VMEM((CW_PAD,), jnp.float32),
        pltpu.VMEM((L,), jnp.float32),
        pltpu.SemaphoreType.DMA,
    ],
)
def _bce_sc(yt_hbm, ypt_hbm, cw_hbm, out_hbm, yt_v, cols_v, cw_v, res_v, sem):
    wid = lax.axis_index("s") * NC + lax.axis_index("c")
    base = wid * CHUNK

    cols_copy = pltpu.async_copy(ypt_hbm.at[:, pl.ds(base, CHUNK)], cols_v, sem)
    pltpu.sync_copy(yt_hbm.at[pl.ds(base, CHUNK)], yt_v)
    cw_v[pl.ds(CW_PAD - L, L)] = jnp.zeros((L,), jnp.float32)
    pltpu.sync_copy(cw_hbm, cw_v.at[pl.ds(0, NCLS)])

    lane = lax.iota(jnp.int32, L)

    sw_vec = cw_v[pl.ds(0, L)]
    for j in range(1, CW_PAD // L):
        sw_vec = sw_vec + cw_v[pl.ds(j * L, L)]
    for s in (8, 4, 2, 1):
        res_v[...] = sw_vec
        sw_vec = sw_vec + plsc.load_gather(res_v, [lane ^ s])
    scale = -1.0 / (sw_vec * float(BATCH))
    cols_copy.wait()

    @plsc.parallel_loop(0, NV, 1, unroll=4, carry=jnp.zeros((L,), jnp.float32))
    def acc(j, acc_in):
        yt16 = yt_v[pl.ds(j * L, L)]
        col16 = lane + j * L
        p16 = plsc.load_gather(cols_v, [yt16, col16])
        w16 = plsc.load_gather(cw_v, [yt16])
        pc = jnp.minimum(jnp.maximum(p16, 0.01), 0.99) + 0.01
        return acc_in + w16 * _log_f32(pc)

    res_v[...] = acc * scale
    pltpu.sync_copy(res_v, out_hbm.at[wid])


def kernel(y_true, y_pred, class_weight):
    yt = y_true.reshape(-1).astype(jnp.int32)
    partials = _bce_sc(yt, y_pred.T, class_weight)
    return jnp.sum(partials)


# speedup vs baseline: 1.0085x; 1.0085x over previous
import functools

import jax
import jax.numpy as jnp
from jax import lax
from jax.experimental import pallas as pl
from jax.experimental.pallas import tpu as pltpu
from jax.experimental.pallas import tpu_sc as plsc

NCLS = 100
BATCH = 16384
NC = 2
NS = 16
L = 16
NW = NC * NS
CHUNK = BATCH // NW
NV = CHUNK // L
CW_PAD = 112

_LN2_HI = 0.693359375
_LN2_LO = -2.12194440e-4
_SQRTHF = 0.70710678118654752440


def _log_f32(x):
    bits = plsc.bitcast(x, jnp.int32)
    e = (bits >> 23) - 126
    m = plsc.bitcast(
        (bits & jnp.int32(0x007FFFFF)) | jnp.int32(0x3F000000), jnp.float32
    )
    below = m < _SQRTHF
    e = jnp.where(below, e - 1, e)
    t = jnp.where(below, m + m - 1.0, m - 1.0)
    ef = e.astype(jnp.float32)

    z = t * t
    p = jnp.full((L,), 7.0376836292e-2, jnp.float32)
    p = p * t - 1.1514610310e-1
    p = p * t + 1.1676998740e-1
    p = p * t - 1.2420140846e-1
    p = p * t + 1.4249322787e-1
    p = p * t - 1.6668057665e-1
    p = p * t + 2.0000714765e-1
    p = p * t - 2.4999993993e-1
    p = p * t + 3.3333331174e-1
    y = t * z * p
    y = y + ef * _LN2_LO
    y = y - 0.5 * z
    return t + y + ef * _LN2_HI


_MESH = plsc.VectorSubcoreMesh(core_axis_name="c", subcore_axis_name="s")


@functools.partial(
    pl.kernel,
    mesh=_MESH,
    compiler_params=pltpu.CompilerParams(
        needs_layout_passes=False, use_tc_tiling_on_sc=True
    ),
    out_type=jax.ShapeDtypeStruct((NW, L), jnp.float32),
    scratch_types=[
        pltpu.VMEM((CHUNK,), jnp.int32),
        pltpu.VMEM((NCLS, CHUNK), jnp.float32),
        pltpu.VMEM((CW_PAD,), jnp.float32),
        pltpu.VMEM((L,), jnp.float32),
        pltpu.SemaphoreType.DMA,
    ],
)
def _bce_sc(yt_hbm, ypt_hbm, cw_hbm, out_hbm, yt_v, cols_v, cw_v, res_v, sem):
    wid = lax.axis_index("s") * NC + lax.axis_index("c")
    base = wid * CHUNK

    cols_copy = pltpu.async_copy(ypt_hbm.at[:, pl.ds(base, CHUNK)], cols_v, sem)
    pltpu.sync_copy(yt_hbm.at[pl.ds(base, CHUNK)], yt_v)
    cw_v[pl.ds(CW_PAD - L, L)] = jnp.zeros((L,), jnp.float32)
    pltpu.sync_copy(cw_hbm, cw_v.at[pl.ds(0, NCLS)])

    lane = lax.iota(jnp.int32, L)

    sw_vec = cw_v[pl.ds(0, L)]
    for j in range(1, CW_PAD // L):
        sw_vec = sw_vec + cw_v[pl.ds(j * L, L)]
    for s in (8, 4, 2, 1):
        res_v[...] = sw_vec
        sw_vec = sw_vec + plsc.load_gather(res_v, [lane ^ s])
    scale = -1.0 / (sw_vec * float(BATCH))
    cols_copy.wait()

    def body(j, acc):
        yt16 = yt_v[pl.ds(j * L, L)]
        col16 = lane + j * L
        p16 = plsc.load_gather(cols_v, [yt16, col16])
        w16 = plsc.load_gather(cw_v, [yt16])
        pc = jnp.minimum(jnp.maximum(p16, 0.01), 0.99) + 0.01
        return acc + w16 * _log_f32(pc)

    acc = lax.fori_loop(0, NV, body, jnp.zeros((L,), jnp.float32))

    res_v[...] = acc * scale
    pltpu.sync_copy(res_v, out_hbm.at[wid])


def kernel(y_true, y_pred, class_weight):
    yt = y_true.reshape(-1).astype(jnp.int32)
    partials = _bce_sc(yt, y_pred.T, class_weight)
    return jnp.sum(partials)


# speedup vs baseline: 1.0145x; 1.0060x over previous
import functools

import jax
import jax.numpy as jnp
from jax import lax
from jax.experimental import pallas as pl
from jax.experimental.pallas import tpu as pltpu
from jax.experimental.pallas import tpu_sc as plsc

NCLS = 100
BATCH = 16384
NC = 2
NS = 16
L = 16
NW = NC * NS
CHUNK = BATCH // NW
NV = CHUNK // L
CW_PAD = 112

_LN2_HI = 0.693359375
_LN2_LO = -2.12194440e-4
_SQRTHF = 0.70710678118654752440


def _log_f32(x):
    bits = plsc.bitcast(x, jnp.int32)
    e = (bits >> 23) - 126
    m = plsc.bitcast(
        (bits & jnp.int32(0x007FFFFF)) | jnp.int32(0x3F000000), jnp.float32
    )
    below = m < _SQRTHF
    e = jnp.where(below, e - 1, e)
    t = jnp.where(below, m + m - 1.0, m - 1.0)
    ef = e.astype(jnp.float32)

    z = t * t
    p = jnp.full((L,), 7.0376836292e-2, jnp.float32)
    p = p * t - 1.1514610310e-1
    p = p * t + 1.1676998740e-1
    p = p * t - 1.2420140846e-1
    p = p * t + 1.4249322787e-1
    p = p * t - 1.6668057665e-1
    p = p * t + 2.0000714765e-1
    p = p * t - 2.4999993993e-1
    p = p * t + 3.3333331174e-1
    y = t * z * p
    y = y + ef * _LN2_LO
    y = y - 0.5 * z
    return t + y + ef * _LN2_HI


_MESH = plsc.VectorSubcoreMesh(core_axis_name="c", subcore_axis_name="s")


@functools.partial(
    pl.kernel,
    mesh=_MESH,
    compiler_params=pltpu.CompilerParams(
        needs_layout_passes=False, use_tc_tiling_on_sc=True
    ),
    out_type=jax.ShapeDtypeStruct((NW // 8, L * 8), jnp.float32),
    scratch_types=[
        pltpu.VMEM((CHUNK,), jnp.int32),
        pltpu.VMEM((NCLS, CHUNK), jnp.float32),
        pltpu.VMEM((CW_PAD,), jnp.float32),
        pltpu.VMEM((L,), jnp.float32),
        pltpu.SemaphoreType.DMA,
    ],
)
def _bce_sc(yt_hbm, ypt_hbm, cw_hbm, out_hbm, yt_v, cols_v, cw_v, res_v, sem):
    wid = lax.axis_index("s") * NC + lax.axis_index("c")
    base = wid * CHUNK

    cols_copy = pltpu.async_copy(ypt_hbm.at[:, pl.ds(base, CHUNK)], cols_v, sem)
    pltpu.sync_copy(yt_hbm.at[pl.ds(base, CHUNK)], yt_v)
    cw_v[pl.ds(CW_PAD - L, L)] = jnp.zeros((L,), jnp.float32)
    pltpu.sync_copy(cw_hbm, cw_v.at[pl.ds(0, NCLS)])

    lane = lax.iota(jnp.int32, L)

    sw_vec = cw_v[pl.ds(0, L)]
    for j in range(1, CW_PAD // L):
        sw_vec = sw_vec + cw_v[pl.ds(j * L, L)]
    for s in (8, 4, 2, 1):
        res_v[...] = sw_vec
        sw_vec = sw_vec + plsc.load_gather(res_v, [lane ^ s])
    scale = -1.0 / (sw_vec * float(BATCH))
    cols_copy.wait()

    def body(j, acc):
        yt16 = yt_v[pl.ds(j * L, L)]
        col16 = lane + j * L
        p16 = plsc.load_gather(cols_v, [yt16, col16])
        w16 = plsc.load_gather(cw_v, [yt16])
        pc = jnp.minimum(jnp.maximum(p16, 0.01), 0.99) + 0.01
        return acc + w16 * _log_f32(pc)

    acc = lax.fori_loop(0, NV, body, jnp.zeros((L,), jnp.float32))

    res_v[...] = acc * scale
    pltpu.sync_copy(res_v, out_hbm.at[wid // 8, pl.ds((wid % 8) * L, L)])


def kernel(y_true, y_pred, class_weight):
    yt = y_true.reshape(-1).astype(jnp.int32)
    partials = _bce_sc(yt, y_pred.T, class_weight)
    return jnp.sum(partials)
